# baseline (speedup 1.0000x reference)
import jax
import jax.numpy as jnp
from jax.experimental import pallas as pl
from jax.experimental.pallas import tpu as pltpu

_N = 2727
_H = 32
_DOUT = 16
_KMAX = 9
_CS = 256


def _lgnn_body(x0_ref, x5_ref, x5t_ref, gat_W_ref, gat_Wt_ref, gat_br_ref,
               gat_bc_ref, lin_W_ref, lin_br_ref, U_ref, bcat_ref,
               last_W_ref, last_br_ref,
               P_ref, lout_ref, xo_ref,
               t_a, t_b, t_c, dinv_ref):
    f32 = jnp.float32
    starts = list(range(0, _N, _CS))

    x6 = jnp.maximum(
        jnp.dot(x5_ref[...], gat_W_ref[...], preferred_element_type=f32)
        + gat_br_ref[...], 0.0)
    x6t = jnp.maximum(
        jnp.dot(gat_Wt_ref[...], x5t_ref[...], preferred_element_type=f32)
        + gat_bc_ref[...], 0.0)
    sh_row = 0.5 * jnp.sum(x6 * x6, axis=1, keepdims=True)
    sh_col = 0.5 * jnp.sum(x6t * x6t, axis=0, keepdims=True)

    for st in starts:
        cs = min(_CS, _N - st)
        g = jnp.dot(x6[st:st + cs], x6t, preferred_element_type=f32)
        p = jnp.exp(g - sh_row[st:st + cs] - sh_col)
        P_ref[st:st + cs, :] = p
        deg = jnp.sum(p, axis=1, keepdims=True) - 1.0
        deg_safe = jnp.where(deg > 0.0, deg, 1.0)
        dinv_ref[st:st + cs, :] = jnp.where(
            deg > 0.0, jax.lax.rsqrt(deg_safe), 0.0)

    dinv = dinv_ref[...]

    x2h = jnp.maximum(
        jnp.dot(x0_ref[...], lin_W_ref[...], preferred_element_type=f32)
        + lin_br_ref[...], 0.0)

    t_a[...] = x2h
    acc = jnp.dot(x2h, U_ref[0], preferred_element_type=f32)

    w = dinv * x2h
    u = jnp.dot(P_ref[...], w, preferred_element_type=f32)
    t1 = -dinv * (u - w)
    t_b[...] = t1
    acc = acc + jnp.dot(t1, U_ref[1], preferred_element_type=f32)

    bufs = [t_a, t_b, t_c]
    for k in range(2, _KMAX):
        prev_r = bufs[(k - 2) % 3]
        cur_r = bufs[(k - 1) % 3]
        nxt_r = bufs[k % 3]
        w = dinv * cur_r[...]
        u = jnp.dot(P_ref[...], w, preferred_element_type=f32)
        t_next = -2.0 * dinv * (u - w) - prev_r[...]
        nxt_r[...] = t_next
        acc = acc + jnp.dot(t_next, U_ref[k], preferred_element_type=f32)

    lout = acc + bcat_ref[...]
    lout_ref[...] = lout
    logits = (jnp.dot(lout, last_W_ref[...], preferred_element_type=f32)
              + last_br_ref[...])
    m = jnp.max(logits, axis=1, keepdims=True)
    e = jnp.exp(logits - m)
    xo_ref[...] = e / jnp.sum(e, axis=1, keepdims=True)


def kernel(data_x_0, data_x_1, data_x_2, data_x_3, data_x_4, data_x_5,
           gat_W, gat_b, lin_W, lin_b, cheb1_W, cheb1_b, cheb2_W, cheb2_b,
           cheb3_W, cheb3_b, last_W, last_b):
    f32 = jnp.float32
    x5 = data_x_5.astype(f32)
    U = jnp.zeros((_KMAX, _H, 3 * _DOUT), f32)
    U = U.at[0:3, :, 0:_DOUT].set(cheb1_W)
    U = U.at[0:6, :, _DOUT:2 * _DOUT].set(cheb2_W)
    U = U.at[0:9, :, 2 * _DOUT:3 * _DOUT].set(cheb3_W)
    bcat = jnp.concatenate([cheb1_b, cheb2_b, cheb3_b])[None, :]

    out_shapes = [
        jax.ShapeDtypeStruct((_N, _N), f32),
        jax.ShapeDtypeStruct((_N, 3 * _DOUT), f32),
        jax.ShapeDtypeStruct((_N, _DOUT), f32),
    ]
    scratch = [
        pltpu.VMEM((_N, _H), f32),
        pltpu.VMEM((_N, _H), f32),
        pltpu.VMEM((_N, _H), f32),
        pltpu.VMEM((_N, 1), f32),
    ]
    P, lout, xo = pl.pallas_call(
        _lgnn_body,
        out_shape=out_shapes,
        scratch_shapes=scratch,
        compiler_params=pltpu.CompilerParams(
            vmem_limit_bytes=100 * 1024 * 1024),
    )(data_x_0, x5, x5.T, gat_W, gat_W.T, gat_b[None, :], gat_b[:, None],
      lin_W, lin_b[None, :], U, bcat, last_W, last_b[None, :])

    return (xo, data_x_3[0], data_x_4[0], P, lout, data_x_0)


# speedup vs baseline: 1.3728x; 1.3728x over previous
import jax
import jax.numpy as jnp
from jax.experimental import pallas as pl
from jax.experimental.pallas import tpu as pltpu

_N = 2727
_H = 32
_DOUT = 16
_KMAX = 9
_CS = 512


def _lgnn_body(x0_ref, x5_ref, x5t_ref, gat_W_ref, gat_Wt_ref, gat_br_ref,
               gat_bc_ref, lin_W_ref, lin_br_ref, U_ref, bcat_ref,
               last_W_ref, last_br_ref,
               P_ref, lout_ref, xo_ref,
               t_a, t_b, t_c, dinv_ref):
    f32 = jnp.float32
    starts = list(range(0, _N, _CS))

    x6 = jnp.maximum(
        jnp.dot(x5_ref[...], gat_W_ref[...], preferred_element_type=f32)
        + gat_br_ref[...], 0.0)
    x6t = jnp.maximum(
        jnp.dot(gat_Wt_ref[...], x5t_ref[...], preferred_element_type=f32)
        + gat_bc_ref[...], 0.0)
    sh_row = 0.5 * jnp.sum(x6 * x6, axis=1, keepdims=True)
    sh_col = 0.5 * jnp.sum(x6t * x6t, axis=0, keepdims=True)

    for st in starts:
        cs = min(_CS, _N - st)
        g = jnp.dot(x6[st:st + cs], x6t, preferred_element_type=f32)
        p = jnp.exp(g - sh_row[st:st + cs] - sh_col)
        P_ref[st:st + cs, :] = p
        deg = jnp.sum(p, axis=1, keepdims=True) - 1.0
        deg_safe = jnp.where(deg > 0.0, deg, 1.0)
        dinv_ref[st:st + cs, :] = jnp.where(
            deg > 0.0, jax.lax.rsqrt(deg_safe), 0.0)

    dinv = dinv_ref[...]

    x2h = jnp.maximum(
        jnp.dot(x0_ref[...], lin_W_ref[...], preferred_element_type=f32)
        + lin_br_ref[...], 0.0)

    t_a[...] = x2h
    acc = jnp.dot(x2h, U_ref[0], preferred_element_type=f32)

    w = dinv * x2h
    for st in starts:
        cs = min(_CS, _N - st)
        u = jnp.dot(P_ref[st:st + cs, :], w, preferred_element_type=f32)
        t_b[st:st + cs, :] = -dinv[st:st + cs] * (u - w[st:st + cs])
    acc = acc + jnp.dot(t_b[...], U_ref[1], preferred_element_type=f32)

    bufs = [t_a, t_b, t_c]
    for k in range(2, _KMAX):
        prev_r = bufs[(k - 2) % 3]
        cur_r = bufs[(k - 1) % 3]
        nxt_r = bufs[k % 3]
        w = dinv * cur_r[...]
        for st in starts:
            cs = min(_CS, _N - st)
            u = jnp.dot(P_ref[st:st + cs, :], w, preferred_element_type=f32)
            lv = -dinv[st:st + cs] * (u - w[st:st + cs])
            nxt_r[st:st + cs, :] = 2.0 * lv - prev_r[st:st + cs, :]
        acc = acc + jnp.dot(nxt_r[...], U_ref[k], preferred_element_type=f32)

    lout = acc + bcat_ref[...]
    lout_ref[...] = lout
    logits = (jnp.dot(lout, last_W_ref[...], preferred_element_type=f32)
              + last_br_ref[...])
    m = jnp.max(logits, axis=1, keepdims=True)
    e = jnp.exp(logits - m)
    xo_ref[...] = e / jnp.sum(e, axis=1, keepdims=True)


def kernel(data_x_0, data_x_1, data_x_2, data_x_3, data_x_4, data_x_5,
           gat_W, gat_b, lin_W, lin_b, cheb1_W, cheb1_b, cheb2_W, cheb2_b,
           cheb3_W, cheb3_b, last_W, last_b):
    f32 = jnp.float32
    x5 = data_x_5.astype(f32)
    U = jnp.zeros((_KMAX, _H, 3 * _DOUT), f32)
    U = U.at[0:3, :, 0:_DOUT].set(cheb1_W)
    U = U.at[0:6, :, _DOUT:2 * _DOUT].set(cheb2_W)
    U = U.at[0:9, :, 2 * _DOUT:3 * _DOUT].set(cheb3_W)
    bcat = jnp.concatenate([cheb1_b, cheb2_b, cheb3_b])[None, :]

    out_shapes = [
        jax.ShapeDtypeStruct((_N, _N), f32),
        jax.ShapeDtypeStruct((_N, 3 * _DOUT), f32),
        jax.ShapeDtypeStruct((_N, _DOUT), f32),
    ]
    scratch = [
        pltpu.VMEM((_N, _H), f32),
        pltpu.VMEM((_N, _H), f32),
        pltpu.VMEM((_N, _H), f32),
        pltpu.VMEM((_N, 1), f32),
    ]
    P, lout, xo = pl.pallas_call(
        _lgnn_body,
        out_shape=out_shapes,
        scratch_shapes=scratch,
        compiler_params=pltpu.CompilerParams(
            vmem_limit_bytes=100 * 1024 * 1024),
    )(data_x_0, x5, x5.T, gat_W, gat_W.T, gat_b[None, :], gat_b[:, None],
      lin_W, lin_b[None, :], U, bcat, last_W, last_b[None, :])

    return (xo, data_x_3[0], data_x_4[0], P, lout, data_x_0)


# speedup vs baseline: 1.3895x; 1.0122x over previous
import jax
import jax.numpy as jnp
from jax.experimental import pallas as pl
from jax.experimental.pallas import tpu as pltpu

_N = 2727
_H = 32
_DOUT = 16
_KMAX = 9
_CS = 256


def _lgnn_body(x0_ref, x5_ref, x5t_ref, gat_W_ref, gat_Wt_ref, gat_br_ref,
               gat_bc_ref, lin_W_ref, lin_br_ref, U_ref, bcat_ref,
               last_W_ref, last_br_ref,
               P_hbm, lout_ref, xo_ref,
               P_ref, t_a, t_b, t_c, dinv_ref, dma_sem):
    f32 = jnp.float32
    starts = list(range(0, _N, _CS))

    def _p_copy(st, cs):
        return pltpu.make_async_copy(
            P_ref.at[pl.ds(st, cs), :], P_hbm.at[pl.ds(st, cs), :], dma_sem)

    x6 = jnp.maximum(
        jnp.dot(x5_ref[...], gat_W_ref[...], preferred_element_type=f32)
        + gat_br_ref[...], 0.0)
    x6t = jnp.maximum(
        jnp.dot(gat_Wt_ref[...], x5t_ref[...], preferred_element_type=f32)
        + gat_bc_ref[...], 0.0)
    sh_row = 0.5 * jnp.sum(x6 * x6, axis=1, keepdims=True)
    sh_col = 0.5 * jnp.sum(x6t * x6t, axis=0, keepdims=True)

    for st in starts:
        cs = min(_CS, _N - st)
        g = jnp.dot(x6[st:st + cs], x6t, preferred_element_type=f32)
        p = jnp.exp(g - sh_row[st:st + cs] - sh_col)
        P_ref[st:st + cs, :] = p
        _p_copy(st, cs).start()
        deg = jnp.sum(p, axis=1, keepdims=True) - 1.0
        deg_safe = jnp.where(deg > 0.0, deg, 1.0)
        dinv_ref[st:st + cs, :] = jnp.where(
            deg > 0.0, jax.lax.rsqrt(deg_safe), 0.0)

    dinv = dinv_ref[...]

    x2h = jnp.maximum(
        jnp.dot(x0_ref[...], lin_W_ref[...], preferred_element_type=f32)
        + lin_br_ref[...], 0.0)

    t_a[...] = x2h
    acc = jnp.dot(x2h, U_ref[0], preferred_element_type=f32)

    w = dinv * x2h
    for st in starts:
        cs = min(_CS, _N - st)
        u = jnp.dot(P_ref[st:st + cs, :], w, preferred_element_type=f32)
        t_b[st:st + cs, :] = -dinv[st:st + cs] * (u - w[st:st + cs])
    acc = acc + jnp.dot(t_b[...], U_ref[1], preferred_element_type=f32)

    bufs = [t_a, t_b, t_c]
    for k in range(2, _KMAX):
        prev_r = bufs[(k - 2) % 3]
        cur_r = bufs[(k - 1) % 3]
        nxt_r = bufs[k % 3]
        w = dinv * cur_r[...]
        for st in starts:
            cs = min(_CS, _N - st)
            u = jnp.dot(P_ref[st:st + cs, :], w, preferred_element_type=f32)
            lv = -dinv[st:st + cs] * (u - w[st:st + cs])
            nxt_r[st:st + cs, :] = 2.0 * lv - prev_r[st:st + cs, :]
        acc = acc + jnp.dot(nxt_r[...], U_ref[k], preferred_element_type=f32)

    lout = acc + bcat_ref[...]
    lout_ref[...] = lout
    logits = (jnp.dot(lout, last_W_ref[...], preferred_element_type=f32)
              + last_br_ref[...])
    m = jnp.max(logits, axis=1, keepdims=True)
    e = jnp.exp(logits - m)
    xo_ref[...] = e / jnp.sum(e, axis=1, keepdims=True)

    for st in starts:
        cs = min(_CS, _N - st)
        _p_copy(st, cs).wait()


def kernel(data_x_0, data_x_1, data_x_2, data_x_3, data_x_4, data_x_5,
           gat_W, gat_b, lin_W, lin_b, cheb1_W, cheb1_b, cheb2_W, cheb2_b,
           cheb3_W, cheb3_b, last_W, last_b):
    f32 = jnp.float32
    x5 = data_x_5.astype(f32)
    U = jnp.zeros((_KMAX, _H, 3 * _DOUT), f32)
    U = U.at[0:3, :, 0:_DOUT].set(cheb1_W)
    U = U.at[0:6, :, _DOUT:2 * _DOUT].set(cheb2_W)
    U = U.at[0:9, :, 2 * _DOUT:3 * _DOUT].set(cheb3_W)
    bcat = jnp.concatenate([cheb1_b, cheb2_b, cheb3_b])[None, :]

    out_shapes = [
        jax.ShapeDtypeStruct((_N, _N), f32),
        jax.ShapeDtypeStruct((_N, 3 * _DOUT), f32),
        jax.ShapeDtypeStruct((_N, _DOUT), f32),
    ]
    scratch = [
        pltpu.VMEM((_N, _N), f32),
        pltpu.VMEM((_N, _H), f32),
        pltpu.VMEM((_N, _H), f32),
        pltpu.VMEM((_N, _H), f32),
        pltpu.VMEM((_N, 1), f32),
        pltpu.SemaphoreType.DMA,
    ]
    P, lout, xo = pl.pallas_call(
        _lgnn_body,
        out_shape=out_shapes,
        out_specs=[
            pl.BlockSpec(memory_space=pl.ANY),
            pl.BlockSpec((_N, 3 * _DOUT), lambda: (0, 0)),
            pl.BlockSpec((_N, _DOUT), lambda: (0, 0)),
        ],
        scratch_shapes=scratch,
        compiler_params=pltpu.CompilerParams(
            vmem_limit_bytes=100 * 1024 * 1024),
    )(data_x_0, x5, x5.T, gat_W, gat_W.T, gat_b[None, :], gat_b[:, None],
      lin_W, lin_b[None, :], U, bcat, last_W, last_b[None, :])

    return (xo, data_x_3[0], data_x_4[0], P, lout, data_x_0)


# speedup vs baseline: 1.3911x; 1.0011x over previous
import jax
import jax.numpy as jnp
from jax.experimental import pallas as pl
from jax.experimental.pallas import tpu as pltpu

_N = 2727
_H = 32
_DOUT = 16
_KMAX = 9
_CS = 256


def _lgnn_body(x0_ref, x5_ref, x5t_ref, gat_W_ref, gat_Wt_ref, gat_br_ref,
               gat_bc_ref, lin_W_ref, lin_br_ref, U_ref, bcat_ref,
               last_W_ref, last_br_ref,
               P_ref, lout_ref, xo_ref,
               Pb_ref, t_a, t_b, t_c, dinv_ref):
    f32 = jnp.float32
    bf16 = jnp.bfloat16
    starts = list(range(0, _N, _CS))

    x6 = jnp.maximum(
        jnp.dot(x5_ref[...], gat_W_ref[...], preferred_element_type=f32)
        + gat_br_ref[...], 0.0)
    x6t = jnp.maximum(
        jnp.dot(gat_Wt_ref[...], x5t_ref[...], preferred_element_type=f32)
        + gat_bc_ref[...], 0.0)
    sh_row = 0.5 * jnp.sum(x6 * x6, axis=1, keepdims=True)
    sh_col = 0.5 * jnp.sum(x6t * x6t, axis=0, keepdims=True)

    for st in starts:
        cs = min(_CS, _N - st)
        g = jnp.dot(x6[st:st + cs], x6t, preferred_element_type=f32)
        p = jnp.exp(g - sh_row[st:st + cs] - sh_col)
        P_ref[st:st + cs, :] = p
        Pb_ref[st:st + cs, :] = p.astype(bf16)
        deg = jnp.sum(p, axis=1, keepdims=True) - 1.0
        deg_safe = jnp.where(deg > 0.0, deg, 1.0)
        dinv_ref[st:st + cs, :] = jnp.where(
            deg > 0.0, jax.lax.rsqrt(deg_safe), 0.0)

    dinv = dinv_ref[...]

    x2h = jnp.maximum(
        jnp.dot(x0_ref[...], lin_W_ref[...], preferred_element_type=f32)
        + lin_br_ref[...], 0.0)

    t_a[...] = x2h
    acc = jnp.dot(x2h, U_ref[0], preferred_element_type=f32)

    w = dinv * x2h
    wb = w.astype(bf16)
    for st in starts:
        cs = min(_CS, _N - st)
        u = jnp.dot(Pb_ref[st:st + cs, :], wb, preferred_element_type=f32)
        t_b[st:st + cs, :] = -dinv[st:st + cs] * (u - w[st:st + cs])
    acc = acc + jnp.dot(t_b[...], U_ref[1], preferred_element_type=f32)

    bufs = [t_a, t_b, t_c]
    for k in range(2, _KMAX):
        prev_r = bufs[(k - 2) % 3]
        cur_r = bufs[(k - 1) % 3]
        nxt_r = bufs[k % 3]
        w = dinv * cur_r[...]
        wb = w.astype(bf16)
        for st in starts:
            cs = min(_CS, _N - st)
            u = jnp.dot(Pb_ref[st:st + cs, :], wb, preferred_element_type=f32)
            lv = -dinv[st:st + cs] * (u - w[st:st + cs])
            nxt_r[st:st + cs, :] = 2.0 * lv - prev_r[st:st + cs, :]
        acc = acc + jnp.dot(nxt_r[...], U_ref[k], preferred_element_type=f32)

    lout = acc + bcat_ref[...]
    lout_ref[...] = lout
    logits = (jnp.dot(lout, last_W_ref[...], preferred_element_type=f32)
              + last_br_ref[...])
    m = jnp.max(logits, axis=1, keepdims=True)
    e = jnp.exp(logits - m)
    xo_ref[...] = e / jnp.sum(e, axis=1, keepdims=True)


def kernel(data_x_0, data_x_1, data_x_2, data_x_3, data_x_4, data_x_5,
           gat_W, gat_b, lin_W, lin_b, cheb1_W, cheb1_b, cheb2_W, cheb2_b,
           cheb3_W, cheb3_b, last_W, last_b):
    f32 = jnp.float32
    x5 = data_x_5.astype(f32)
    U = jnp.zeros((_KMAX, _H, 3 * _DOUT), f32)
    U = U.at[0:3, :, 0:_DOUT].set(cheb1_W)
    U = U.at[0:6, :, _DOUT:2 * _DOUT].set(cheb2_W)
    U = U.at[0:9, :, 2 * _DOUT:3 * _DOUT].set(cheb3_W)
    bcat = jnp.concatenate([cheb1_b, cheb2_b, cheb3_b])[None, :]

    out_shapes = [
        jax.ShapeDtypeStruct((_N, _N), f32),
        jax.ShapeDtypeStruct((_N, 3 * _DOUT), f32),
        jax.ShapeDtypeStruct((_N, _DOUT), f32),
    ]
    scratch = [
        pltpu.VMEM((_N, _N), jnp.bfloat16),
        pltpu.VMEM((_N, _H), f32),
        pltpu.VMEM((_N, _H), f32),
        pltpu.VMEM((_N, _H), f32),
        pltpu.VMEM((_N, 1), f32),
    ]
    P, lout, xo = pl.pallas_call(
        _lgnn_body,
        out_shape=out_shapes,
        scratch_shapes=scratch,
        compiler_params=pltpu.CompilerParams(
            vmem_limit_bytes=100 * 1024 * 1024),
    )(data_x_0, x5, x5.T, gat_W, gat_W.T, gat_b[None, :], gat_b[:, None],
      lin_W, lin_b[None, :], U, bcat, last_W, last_b[None, :])

    return (xo, data_x_3[0], data_x_4[0], P, lout, data_x_0)


# speedup vs baseline: 1.3949x; 1.0027x over previous
import jax
import jax.numpy as jnp
from jax.experimental import pallas as pl
from jax.experimental.pallas import tpu as pltpu

_N = 2727
_H = 32
_DOUT = 16
_KMAX = 9
_CS = 128


def _lgnn_body(x0_ref, x5_ref, x5t_ref, gat_W_ref, gat_Wt_ref, gat_br_ref,
               gat_bc_ref, lin_W_ref, lin_br_ref, U_ref, bcat_ref,
               last_W_ref, last_br_ref,
               P_ref, lout_ref, xo_ref,
               t_a, t_b, t_c, dinv_ref):
    f32 = jnp.float32
    starts = list(range(0, _N, _CS))

    x6 = jnp.maximum(
        jnp.dot(x5_ref[...], gat_W_ref[...], preferred_element_type=f32)
        + gat_br_ref[...], 0.0)
    x6t = jnp.maximum(
        jnp.dot(gat_Wt_ref[...], x5t_ref[...], preferred_element_type=f32)
        + gat_bc_ref[...], 0.0)
    sh_row = 0.5 * jnp.sum(x6 * x6, axis=1, keepdims=True)
    sh_col = 0.5 * jnp.sum(x6t * x6t, axis=0, keepdims=True)

    for st in starts:
        cs = min(_CS, _N - st)
        g = jnp.dot(x6[st:st + cs], x6t, preferred_element_type=f32)
        p = jnp.exp(g - sh_row[st:st + cs] - sh_col)
        P_ref[st:st + cs, :] = p
        deg = jnp.sum(p, axis=1, keepdims=True) - 1.0
        deg_safe = jnp.where(deg > 0.0, deg, 1.0)
        dinv_ref[st:st + cs, :] = jnp.where(
            deg > 0.0, jax.lax.rsqrt(deg_safe), 0.0)

    dinv = dinv_ref[...]

    x2h = jnp.maximum(
        jnp.dot(x0_ref[...], lin_W_ref[...], preferred_element_type=f32)
        + lin_br_ref[...], 0.0)

    t_a[...] = x2h
    acc = jnp.dot(x2h, U_ref[0], preferred_element_type=f32)

    w = dinv * x2h
    for st in starts:
        cs = min(_CS, _N - st)
        u = jnp.dot(P_ref[st:st + cs, :], w, preferred_element_type=f32)
        t_b[st:st + cs, :] = -dinv[st:st + cs] * (u - w[st:st + cs])
    acc = acc + jnp.dot(t_b[...], U_ref[1], preferred_element_type=f32)

    bufs = [t_a, t_b, t_c]
    for k in range(2, _KMAX):
        prev_r = bufs[(k - 2) % 3]
        cur_r = bufs[(k - 1) % 3]
        nxt_r = bufs[k % 3]
        w = dinv * cur_r[...]
        for st in starts:
            cs = min(_CS, _N - st)
            u = jnp.dot(P_ref[st:st + cs, :], w, preferred_element_type=f32)
            lv = -dinv[st:st + cs] * (u - w[st:st + cs])
            nxt_r[st:st + cs, :] = 2.0 * lv - prev_r[st:st + cs, :]
        acc = acc + jnp.dot(nxt_r[...], U_ref[k], preferred_element_type=f32)

    lout = acc + bcat_ref[...]
    lout_ref[...] = lout
    logits = (jnp.dot(lout, last_W_ref[...], preferred_element_type=f32)
              + last_br_ref[...])
    m = jnp.max(logits, axis=1, keepdims=True)
    e = jnp.exp(logits - m)
    xo_ref[...] = e / jnp.sum(e, axis=1, keepdims=True)


def kernel(data_x_0, data_x_1, data_x_2, data_x_3, data_x_4, data_x_5,
           gat_W, gat_b, lin_W, lin_b, cheb1_W, cheb1_b, cheb2_W, cheb2_b,
           cheb3_W, cheb3_b, last_W, last_b):
    f32 = jnp.float32
    x5 = data_x_5.astype(f32)
    U = jnp.zeros((_KMAX, _H, 3 * _DOUT), f32)
    U = U.at[0:3, :, 0:_DOUT].set(cheb1_W)
    U = U.at[0:6, :, _DOUT:2 * _DOUT].set(cheb2_W)
    U = U.at[0:9, :, 2 * _DOUT:3 * _DOUT].set(cheb3_W)
    bcat = jnp.concatenate([cheb1_b, cheb2_b, cheb3_b])[None, :]

    out_shapes = [
        jax.ShapeDtypeStruct((_N, _N), f32),
        jax.ShapeDtypeStruct((_N, 3 * _DOUT), f32),
        jax.ShapeDtypeStruct((_N, _DOUT), f32),
    ]
    scratch = [
        pltpu.VMEM((_N, _H), f32),
        pltpu.VMEM((_N, _H), f32),
        pltpu.VMEM((_N, _H), f32),
        pltpu.VMEM((_N, 1), f32),
    ]
    P, lout, xo = pl.pallas_call(
        _lgnn_body,
        out_shape=out_shapes,
        scratch_shapes=scratch,
        compiler_params=pltpu.CompilerParams(
            vmem_limit_bytes=100 * 1024 * 1024),
    )(data_x_0, x5, x5.T, gat_W, gat_W.T, gat_b[None, :], gat_b[:, None],
      lin_W, lin_b[None, :], U, bcat, last_W, last_b[None, :])

    return (xo, data_x_3[0], data_x_4[0], P, lout, data_x_0)


# speedup vs baseline: 1.4139x; 1.0136x over previous
import jax
import jax.numpy as jnp
from jax.experimental import pallas as pl
from jax.experimental.pallas import tpu as pltpu

_N = 2727
_H = 32
_DOUT = 16
_KMAX = 9
_CS = 256


def _lgnn_body(x0_ref, x5_ref, x5t_ref, gat_W_ref, gat_Wt_ref, gat_br_ref,
               gat_bc_ref, lin_W_ref, lin_br_ref, U_ref, bcat_ref,
               last_W_ref, last_br_ref,
               P_ref, lout_ref, xo_ref,
               t_a, t_b, t_c, dinv_ref):
    f32 = jnp.float32
    starts = list(range(0, _N, _CS))

    x6 = jnp.maximum(
        jnp.dot(x5_ref[...], gat_W_ref[...], preferred_element_type=f32)
        + gat_br_ref[...], 0.0)
    x6t = jnp.maximum(
        jnp.dot(gat_Wt_ref[...], x5t_ref[...], preferred_element_type=f32)
        + gat_bc_ref[...], 0.0)
    sh_row = 0.5 * jnp.sum(x6 * x6, axis=1, keepdims=True)
    sh_col = 0.5 * jnp.sum(x6t * x6t, axis=0, keepdims=True)

    for st in starts:
        cs = min(_CS, _N - st)
        g = jnp.dot(x6[st:st + cs], x6t, preferred_element_type=f32)
        p = jnp.exp(g - sh_row[st:st + cs] - sh_col)
        P_ref[st:st + cs, :] = p
        deg = jnp.sum(p, axis=1, keepdims=True) - 1.0
        deg_safe = jnp.where(deg > 0.0, deg, 1.0)
        dinv_ref[st:st + cs, :] = jnp.where(
            deg > 0.0, jax.lax.rsqrt(deg_safe), 0.0)

    dinv = dinv_ref[...]

    x2h = jnp.maximum(
        jnp.dot(x0_ref[...], lin_W_ref[...], preferred_element_type=f32)
        + lin_br_ref[...], 0.0)

    t_a[...] = x2h
    acc = jnp.dot(x2h, U_ref[0], preferred_element_type=f32)

    w = dinv * x2h
    for st in starts:
        cs = min(_CS, _N - st)
        u = jnp.dot(P_ref[st:st + cs, :], w, preferred_element_type=f32)
        t_b[st:st + cs, :] = -dinv[st:st + cs] * (u - w[st:st + cs])
    acc = acc + jnp.dot(t_b[...], U_ref[1], preferred_element_type=f32)

    bufs = [t_a, t_b, t_c]
    for k in range(2, _KMAX):
        prev_r = bufs[(k - 2) % 3]
        cur_r = bufs[(k - 1) % 3]
        nxt_r = bufs[k % 3]
        w = dinv * cur_r[...]
        for st in starts:
            cs = min(_CS, _N - st)
            u = jnp.dot(P_ref[st:st + cs, :], w, preferred_element_type=f32)
            lv = -dinv[st:st + cs] * (u - w[st:st + cs])
            nxt_r[st:st + cs, :] = 2.0 * lv - prev_r[st:st + cs, :]
        acc = acc + jnp.dot(nxt_r[...], U_ref[k], preferred_element_type=f32)

    lout = acc + bcat_ref[...]
    lout_ref[...] = lout
    logits = (jnp.dot(lout, last_W_ref[...], preferred_element_type=f32)
              + last_br_ref[...])
    m = jnp.max(logits, axis=1, keepdims=True)
    e = jnp.exp(logits - m)
    xo_ref[...] = e / jnp.sum(e, axis=1, keepdims=True)


def kernel(data_x_0, data_x_1, data_x_2, data_x_3, data_x_4, data_x_5,
           gat_W, gat_b, lin_W, lin_b, cheb1_W, cheb1_b, cheb2_W, cheb2_b,
           cheb3_W, cheb3_b, last_W, last_b):
    f32 = jnp.float32
    x5 = data_x_5.astype(f32)
    U = jnp.zeros((_KMAX, _H, 3 * _DOUT), f32)
    U = U.at[0:3, :, 0:_DOUT].set(cheb1_W)
    U = U.at[0:6, :, _DOUT:2 * _DOUT].set(cheb2_W)
    U = U.at[0:9, :, 2 * _DOUT:3 * _DOUT].set(cheb3_W)
    bcat = jnp.concatenate([cheb1_b, cheb2_b, cheb3_b])[None, :]

    out_shapes = [
        jax.ShapeDtypeStruct((_N, _N), f32),
        jax.ShapeDtypeStruct((_N, 3 * _DOUT), f32),
        jax.ShapeDtypeStruct((_N, _DOUT), f32),
    ]
    scratch = [
        pltpu.VMEM((_N, _H), f32),
        pltpu.VMEM((_N, _H), f32),
        pltpu.VMEM((_N, _H), f32),
        pltpu.VMEM((_N, 1), f32),
    ]
    P, lout, xo = pl.pallas_call(
        _lgnn_body,
        out_shape=out_shapes,
        scratch_shapes=scratch,
        compiler_params=pltpu.CompilerParams(
            vmem_limit_bytes=100 * 1024 * 1024),
    )(data_x_0, x5, x5.T, gat_W, gat_W.T, gat_b[None, :], gat_b[:, None],
      lin_W, lin_b[None, :], U, bcat, last_W, last_b[None, :])

    return (xo, data_x_3[0], data_x_4[0], P, lout, data_x_0)


# speedup vs baseline: 1.4335x; 1.0138x over previous
import jax
import jax.numpy as jnp
from jax.experimental import pallas as pl
from jax.experimental.pallas import tpu as pltpu

_N = 2727
_H = 32
_DOUT = 16
_KMAX = 9
_CS = 256


def _lgnn_body(x0_ref, x5_ref, x5t_ref, gat_W_ref, gat_Wt_ref, gat_br_ref,
               gat_bc_ref, lin_W_ref, lin_br_ref, U_ref, bcat_ref,
               last_W_ref, last_br_ref,
               P_ref, lout_ref, xo_ref,
               t_a, t_b, t_c, dinv_ref):
    f32 = jnp.float32
    starts = list(range(0, _N, _CS))

    x6 = jnp.maximum(
        jnp.dot(x5_ref[...], gat_W_ref[...], preferred_element_type=f32)
        + gat_br_ref[...], 0.0)
    x6t = jnp.maximum(
        jnp.dot(gat_Wt_ref[...], x5t_ref[...], preferred_element_type=f32)
        + gat_bc_ref[...], 0.0)
    sh_row = 0.5 * jnp.sum(x6 * x6, axis=1, keepdims=True)
    sh_col = 0.5 * jnp.sum(x6t * x6t, axis=0, keepdims=True)
    x6aug = jnp.concatenate(
        [x6, jnp.ones((_N, 1), f32), sh_row], axis=1)
    x6taug = jnp.concatenate(
        [x6t, -sh_col, jnp.full((1, _N), -1.0, f32)], axis=0)

    for st in starts:
        cs = min(_CS, _N - st)
        arg = jnp.dot(x6aug[st:st + cs], x6taug, preferred_element_type=f32)
        p = jnp.exp(arg)
        P_ref[st:st + cs, :] = p
        deg = jnp.sum(p, axis=1, keepdims=True) - 1.0
        deg_safe = jnp.where(deg > 0.0, deg, 1.0)
        dinv_ref[st:st + cs, :] = jnp.where(
            deg > 0.0, jax.lax.rsqrt(deg_safe), 0.0)

    dinv = dinv_ref[...]

    x2h = jnp.maximum(
        jnp.dot(x0_ref[...], lin_W_ref[...], preferred_element_type=f32)
        + lin_br_ref[...], 0.0)

    t_a[...] = x2h
    acc = jnp.dot(x2h, U_ref[0], preferred_element_type=f32)

    w = dinv * x2h
    for st in starts:
        cs = min(_CS, _N - st)
        u = jnp.dot(P_ref[st:st + cs, :], w, preferred_element_type=f32)
        t_b[st:st + cs, :] = -dinv[st:st + cs] * (u - w[st:st + cs])
    acc = acc + jnp.dot(t_b[...], U_ref[1], preferred_element_type=f32)

    bufs = [t_a, t_b, t_c]
    for k in range(2, _KMAX):
        prev_r = bufs[(k - 2) % 3]
        cur_r = bufs[(k - 1) % 3]
        nxt_r = bufs[k % 3]
        w = dinv * cur_r[...]
        for st in starts:
            cs = min(_CS, _N - st)
            u = jnp.dot(P_ref[st:st + cs, :], w, preferred_element_type=f32)
            lv = -dinv[st:st + cs] * (u - w[st:st + cs])
            nxt_r[st:st + cs, :] = 2.0 * lv - prev_r[st:st + cs, :]
        acc = acc + jnp.dot(nxt_r[...], U_ref[k], preferred_element_type=f32)

    lout = acc + bcat_ref[...]
    lout_ref[...] = lout
    logits = (jnp.dot(lout, last_W_ref[...], preferred_element_type=f32)
              + last_br_ref[...])
    m = jnp.max(logits, axis=1, keepdims=True)
    e = jnp.exp(logits - m)
    xo_ref[...] = e / jnp.sum(e, axis=1, keepdims=True)


def kernel(data_x_0, data_x_1, data_x_2, data_x_3, data_x_4, data_x_5,
           gat_W, gat_b, lin_W, lin_b, cheb1_W, cheb1_b, cheb2_W, cheb2_b,
           cheb3_W, cheb3_b, last_W, last_b):
    f32 = jnp.float32
    x5 = data_x_5.astype(f32)
    U = jnp.zeros((_KMAX, _H, 3 * _DOUT), f32)
    U = U.at[0:3, :, 0:_DOUT].set(cheb1_W)
    U = U.at[0:6, :, _DOUT:2 * _DOUT].set(cheb2_W)
    U = U.at[0:9, :, 2 * _DOUT:3 * _DOUT].set(cheb3_W)
    bcat = jnp.concatenate([cheb1_b, cheb2_b, cheb3_b])[None, :]

    out_shapes = [
        jax.ShapeDtypeStruct((_N, _N), f32),
        jax.ShapeDtypeStruct((_N, 3 * _DOUT), f32),
        jax.ShapeDtypeStruct((_N, _DOUT), f32),
    ]
    scratch = [
        pltpu.VMEM((_N, _H), f32),
        pltpu.VMEM((_N, _H), f32),
        pltpu.VMEM((_N, _H), f32),
        pltpu.VMEM((_N, 1), f32),
    ]
    P, lout, xo = pl.pallas_call(
        _lgnn_body,
        out_shape=out_shapes,
        scratch_shapes=scratch,
        compiler_params=pltpu.CompilerParams(
            vmem_limit_bytes=100 * 1024 * 1024),
    )(data_x_0, x5, x5.T, gat_W, gat_W.T, gat_b[None, :], gat_b[:, None],
      lin_W, lin_b[None, :], U, bcat, last_W, last_b[None, :])

    return (xo, data_x_3[0], data_x_4[0], P, lout, data_x_0)


# speedup vs baseline: 1.4401x; 1.0047x over previous
import jax
import jax.numpy as jnp
from jax.experimental import pallas as pl
from jax.experimental.pallas import tpu as pltpu

_N = 2727
_H = 32
_DOUT = 16
_KMAX = 9
_CS = 256


def _lgnn_body(x0_ref, x5_ref, x5t_ref, gat_W_ref, gat_Wt_ref, gat_br_ref,
               gat_bc_ref, lin_W_ref, lin_br_ref, U_ref, bcat_ref,
               last_W_ref, last_br_ref,
               P_ref, lout_ref, xo_ref,
               dinv_ref):
    f32 = jnp.float32
    starts = list(range(0, _N, _CS))

    x6 = jnp.maximum(
        jnp.dot(x5_ref[...], gat_W_ref[...], preferred_element_type=f32)
        + gat_br_ref[...], 0.0)
    x6t = jnp.maximum(
        jnp.dot(gat_Wt_ref[...], x5t_ref[...], preferred_element_type=f32)
        + gat_bc_ref[...], 0.0)
    sh_row = 0.5 * jnp.sum(x6 * x6, axis=1, keepdims=True)
    sh_col = 0.5 * jnp.sum(x6t * x6t, axis=0, keepdims=True)
    x6aug = jnp.concatenate(
        [x6, jnp.ones((_N, 1), f32), sh_row], axis=1)
    x6taug = jnp.concatenate(
        [x6t, -sh_col, jnp.full((1, _N), -1.0, f32)], axis=0)

    for st in starts:
        cs = min(_CS, _N - st)
        arg = jnp.dot(x6aug[st:st + cs], x6taug, preferred_element_type=f32)
        p = jnp.exp(arg)
        P_ref[st:st + cs, :] = p
        deg = jnp.sum(p, axis=1, keepdims=True) - 1.0
        deg_safe = jnp.where(deg > 0.0, deg, 1.0)
        dinv_ref[st:st + cs, :] = jnp.where(
            deg > 0.0, jax.lax.rsqrt(deg_safe), 0.0)

    dinv = dinv_ref[...]

    x2h = jnp.maximum(
        jnp.dot(x0_ref[...], lin_W_ref[...], preferred_element_type=f32)
        + lin_br_ref[...], 0.0)

    acc = jnp.dot(x2h, U_ref[0], preferred_element_type=f32)

    w = dinv * x2h
    parts = []
    for st in starts:
        cs = min(_CS, _N - st)
        u = jnp.dot(P_ref[st:st + cs, :], w, preferred_element_type=f32)
        parts.append(-dinv[st:st + cs] * (u - w[st:st + cs]))
    cur = jnp.concatenate(parts, axis=0)
    prev = x2h
    acc = acc + jnp.dot(cur, U_ref[1], preferred_element_type=f32)

    for k in range(2, _KMAX):
        w = dinv * cur
        parts = []
        for st in starts:
            cs = min(_CS, _N - st)
            u = jnp.dot(P_ref[st:st + cs, :], w, preferred_element_type=f32)
            lv = -dinv[st:st + cs] * (u - w[st:st + cs])
            parts.append(2.0 * lv - prev[st:st + cs, :])
        nxt = jnp.concatenate(parts, axis=0)
        acc = acc + jnp.dot(nxt, U_ref[k], preferred_element_type=f32)
        prev, cur = cur, nxt

    lout = acc + bcat_ref[...]
    lout_ref[...] = lout
    logits = (jnp.dot(lout, last_W_ref[...], preferred_element_type=f32)
              + last_br_ref[...])
    m = jnp.max(logits, axis=1, keepdims=True)
    e = jnp.exp(logits - m)
    xo_ref[...] = e / jnp.sum(e, axis=1, keepdims=True)


def kernel(data_x_0, data_x_1, data_x_2, data_x_3, data_x_4, data_x_5,
           gat_W, gat_b, lin_W, lin_b, cheb1_W, cheb1_b, cheb2_W, cheb2_b,
           cheb3_W, cheb3_b, last_W, last_b):
    f32 = jnp.float32
    x5 = data_x_5.astype(f32)
    U = jnp.zeros((_KMAX, _H, 3 * _DOUT), f32)
    U = U.at[0:3, :, 0:_DOUT].set(cheb1_W)
    U = U.at[0:6, :, _DOUT:2 * _DOUT].set(cheb2_W)
    U = U.at[0:9, :, 2 * _DOUT:3 * _DOUT].set(cheb3_W)
    bcat = jnp.concatenate([cheb1_b, cheb2_b, cheb3_b])[None, :]

    out_shapes = [
        jax.ShapeDtypeStruct((_N, _N), f32),
        jax.ShapeDtypeStruct((_N, 3 * _DOUT), f32),
        jax.ShapeDtypeStruct((_N, _DOUT), f32),
    ]
    scratch = [
        pltpu.VMEM((_N, 1), f32),
    ]
    P, lout, xo = pl.pallas_call(
        _lgnn_body,
        out_shape=out_shapes,
        scratch_shapes=scratch,
        compiler_params=pltpu.CompilerParams(
            vmem_limit_bytes=100 * 1024 * 1024),
    )(data_x_0, x5, x5.T, gat_W, gat_W.T, gat_b[None, :], gat_b[:, None],
      lin_W, lin_b[None, :], U, bcat, last_W, last_b[None, :])

    return (xo, data_x_3[0], data_x_4[0], P, lout, data_x_0)
